# baseline (speedup 1.0000x reference)
import functools

import jax
import jax.numpy as jnp
from jax import lax
from jax.experimental import pallas as pl
from jax.experimental.pallas import tpu as pltpu
from jax.experimental.pallas import tpu_sc as plsc

F = 26
V = 100000
D = 32
B = 16384
BF = B * F

NC = 2
NS = 16
NW = NC * NS
PER_W = BF // NW
CH = 128
N_CH = PER_W // CH


def _sc_gather(table2d, idx_flat):
    mesh = plsc.VectorSubcoreMesh(core_axis_name="c", subcore_axis_name="s")

    @functools.partial(
        pl.kernel,
        mesh=mesh,
        compiler_params=pltpu.CompilerParams(use_tc_tiling_on_sc=False),
        out_type=jax.ShapeDtypeStruct((BF, D), jnp.float32),
        scratch_types=[
            pltpu.VMEM((PER_W,), jnp.int32),
            pltpu.VMEM((CH, D), jnp.float32),
            pltpu.SemaphoreType.DMA,
        ],
    )
    def k(table_hbm, idx_hbm, out_hbm, idx_v, rows_v, sem):
        wid = lax.axis_index("s") * NC + lax.axis_index("c")
        base = pl.multiple_of(wid * PER_W, 8)
        pltpu.sync_copy(idx_hbm.at[pl.ds(base, PER_W)], idx_v)

        def body(i, carry):
            off = pl.multiple_of(i * CH, 8)
            pltpu.async_copy(
                table_hbm.at[idx_v.at[pl.ds(off, CH)]], rows_v, sem
            ).wait()
            pltpu.sync_copy(rows_v, out_hbm.at[pl.ds(base + off, CH)])
            return carry

        lax.fori_loop(0, N_CH, body, 0)

    return k(table2d, idx_flat)


def _tc_head(emb2d, W1, b1, W2, b2, W3, b3, S):
    bs = 1024
    d_in = F * D

    def body(emb_ref, w1_ref, b1_ref, w2_ref, b2_ref, w3_ref, b3_ref,
             s_ref, out_ref):
        flat = emb_ref[...]
        s = s_ref[...]
        ssum = jnp.dot(flat, s, preferred_element_type=jnp.float32)
        ssq = jnp.dot(flat * flat, s, preferred_element_type=jnp.float32)
        fm = 0.5 * jnp.sum(ssum * ssum - ssq, axis=1, keepdims=True)
        h = jnp.maximum(
            jnp.dot(flat, w1_ref[...], preferred_element_type=jnp.float32)
            + b1_ref[...], 0.0)
        h = jnp.maximum(
            jnp.dot(h, w2_ref[...], preferred_element_type=jnp.float32)
            + b2_ref[...], 0.0)
        h = jnp.maximum(
            jnp.dot(h, w3_ref[...], preferred_element_type=jnp.float32)
            + b3_ref[...], 0.0)
        out_ref[...] = fm + h

    full = lambda shape: pl.BlockSpec(shape, lambda i: (0, 0))
    return pl.pallas_call(
        body,
        grid=(B // bs,),
        in_specs=[
            pl.BlockSpec((bs, d_in), lambda i: (i, 0)),
            full((d_in, 128)),
            full((1, 128)),
            full((128, 16)),
            full((1, 16)),
            full((16, 2)),
            full((1, 2)),
            full((d_in, D)),
        ],
        out_specs=pl.BlockSpec((bs, 2), lambda i: (i, 0)),
        out_shape=jax.ShapeDtypeStruct((B, 2), jnp.float32),
    )(emb2d, W1, b1, W2, b2, W3, b3, S)


def kernel(x, tables, W1, b1, W2, b2, W3, b3):
    x = x.astype(jnp.int32)
    idx_flat = (x + (jnp.arange(F, dtype=jnp.int32) * V)[None, :]).reshape(BF)
    table2d = tables.reshape(F * V, D)
    emb = _sc_gather(table2d, idx_flat)
    emb2d = emb.reshape(B, F * D)
    S = jnp.tile(jnp.eye(D, dtype=jnp.float32), (F, 1))
    return _tc_head(emb2d, W1, b1.reshape(1, -1), W2, b2.reshape(1, -1),
                    W3, b3.reshape(1, -1), S)


# speedup vs baseline: 2.9465x; 2.9465x over previous
import functools

import jax
import jax.numpy as jnp
from jax import lax
from jax.experimental import pallas as pl
from jax.experimental.pallas import tpu as pltpu
from jax.experimental.pallas import tpu_sc as plsc

F = 26
V = 100000
D = 32
B = 16384
FD = F * D

NC = 2
NS = 16
NW = NC * NS
HALF = B // 2


def _sc_gather_t(tables2, xt):
    mesh = plsc.VectorSubcoreMesh(core_axis_name="c", subcore_axis_name="s")

    @functools.partial(
        pl.kernel,
        mesh=mesh,
        compiler_params=pltpu.CompilerParams(
            use_tc_tiling_on_sc=True, needs_layout_passes=False),
        out_type=jax.ShapeDtypeStruct((FD, B), jnp.float32),
        scratch_types=[
            pltpu.VMEM((B,), jnp.int32),
            pltpu.VMEM((V,), jnp.float32),
            pltpu.VMEM((HALF,), jnp.float32),
        ],
    )
    def k(tab_hbm, xt_hbm, out_hbm, idx_v, row_v, obuf_v):
        wid = lax.axis_index("s") * NC + lax.axis_index("c")

        def step(kf, carry):
            r = kf * D + wid
            pltpu.sync_copy(xt_hbm.at[kf, :], idx_v)
            pltpu.sync_copy(tab_hbm.at[r, :], row_v)

            def half(h, c2):
                base = h * HALF

                def jloop(j, c3):
                    o = j * 16
                    vidx = idx_v[pl.ds(base + o, 16)]
                    vals = plsc.load_gather(row_v, [vidx])
                    obuf_v[pl.ds(o, 16)] = vals
                    return c3

                lax.fori_loop(0, HALF // 16, jloop, 0)
                pltpu.sync_copy(obuf_v, out_hbm.at[r, pl.ds(base, HALF)])
                return c2

            lax.fori_loop(0, 2, half, 0)
            return carry

        lax.fori_loop(0, F, step, 0)

    return k(tables2, xt)


def _tc_head_t(embt, W1t, b1, W2t, b2, W3t, b3, St):
    bs = 2048

    def body(e_ref, w1_ref, b1_ref, w2_ref, b2_ref, w3_ref, b3_ref,
             st_ref, out_ref):
        e = e_ref[...]
        st = st_ref[...]
        ssum = jnp.dot(st, e, preferred_element_type=jnp.float32)
        ssq = jnp.dot(st, e * e, preferred_element_type=jnp.float32)
        fm = 0.5 * jnp.sum(ssum * ssum - ssq, axis=0, keepdims=True)
        h = jnp.maximum(
            jnp.dot(w1_ref[...], e, preferred_element_type=jnp.float32)
            + b1_ref[...], 0.0)
        h = jnp.maximum(
            jnp.dot(w2_ref[...], h, preferred_element_type=jnp.float32)
            + b2_ref[...], 0.0)
        h = jnp.maximum(
            jnp.dot(w3_ref[...], h, preferred_element_type=jnp.float32)
            + b3_ref[...], 0.0)
        out_ref[...] = fm + h

    full = lambda shape: pl.BlockSpec(shape, lambda i: (0, 0))
    return pl.pallas_call(
        body,
        grid=(B // bs,),
        in_specs=[
            pl.BlockSpec((FD, bs), lambda i: (0, i)),
            full((128, FD)),
            full((128, 1)),
            full((16, 128)),
            full((16, 1)),
            full((2, 16)),
            full((2, 1)),
            full((D, FD)),
        ],
        out_specs=pl.BlockSpec((2, bs), lambda i: (0, i)),
        out_shape=jax.ShapeDtypeStruct((2, B), jnp.float32),
    )(embt, W1t, b1, W2t, b2, W3t, b3, St)


def kernel(x, tables, W1, b1, W2, b2, W3, b3):
    xt = x.astype(jnp.int32).T
    tables2 = tables.transpose(0, 2, 1).reshape(FD, V)
    embt = _sc_gather_t(tables2, xt)
    St = jnp.tile(jnp.eye(D, dtype=jnp.float32), (1, F))
    outt = _tc_head_t(embt, W1.T, b1.reshape(-1, 1), W2.T, b2.reshape(-1, 1),
                      W3.T, b3.reshape(-1, 1), St)
    return outt.T


# speedup vs baseline: 3.1121x; 1.0562x over previous
import functools

import jax
import jax.numpy as jnp
from jax import lax
from jax.experimental import pallas as pl
from jax.experimental.pallas import tpu as pltpu
from jax.experimental.pallas import tpu_sc as plsc

F = 26
V = 100000
D = 32
B = 16384
FD = F * D

NC = 2
NS = 16
NW = NC * NS
QUART = B // 4


def _sc_gather_t(tables2, xt):
    mesh = plsc.VectorSubcoreMesh(core_axis_name="c", subcore_axis_name="s")

    @functools.partial(
        pl.kernel,
        mesh=mesh,
        compiler_params=pltpu.CompilerParams(
            use_tc_tiling_on_sc=True, needs_layout_passes=False),
        out_type=jax.ShapeDtypeStruct((FD, B), jnp.float32),
        scratch_types=[
            pltpu.VMEM((B,), jnp.int32),
            pltpu.VMEM((V,), jnp.float32),
            pltpu.VMEM((QUART,), jnp.float32),
            pltpu.VMEM((QUART,), jnp.float32),
            pltpu.SemaphoreType.DMA,
            pltpu.SemaphoreType.DMA,
            pltpu.SemaphoreType.DMA,
        ],
    )
    def k(tab_hbm, xt_hbm, out_hbm, idx_v, row_v, ob0, ob1, sem_in,
          sem_o0, sem_o1):
        wid = lax.axis_index("s") * NC + lax.axis_index("c")
        obufs = (ob0, ob1)
        osems = (sem_o0, sem_o1)
        pending = [None, None]

        U = 4

        for kf in range(F):
            r = kf * D + wid
            c_idx = pltpu.async_copy(xt_hbm.at[kf, :], idx_v, sem_in)
            c_row = pltpu.async_copy(tab_hbm.at[r, :], row_v, sem_in)
            c_idx.wait()
            c_row.wait()

            for q in range(4):
                base = q * QUART
                ob = obufs[q % 2]
                if pending[q % 2] is not None:
                    pending[q % 2].wait()

                def jloop(j, c3, base=base, ob=ob):
                    o = j * (16 * U)
                    for u in range(U):
                        oo = o + u * 16
                        vidx = idx_v[pl.ds(base + oo, 16)]
                        vals = plsc.load_gather(row_v, [vidx])
                        ob[pl.ds(oo, 16)] = vals
                    return c3

                lax.fori_loop(0, QUART // (16 * U), jloop, 0)
                pending[q % 2] = pltpu.async_copy(
                    ob, out_hbm.at[r, pl.ds(base, QUART)], osems[q % 2])

        pending[0].wait()
        pending[1].wait()

    return k(tables2, xt)


def _tc_head_t(embt, W1t, b1, W2t, b2, W3t, b3, St):
    bs = 2048

    def body(e_ref, w1_ref, b1_ref, w2_ref, b2_ref, w3_ref, b3_ref,
             st_ref, out_ref):
        e = e_ref[...]
        st = st_ref[...]
        ssum = jnp.dot(st, e, preferred_element_type=jnp.float32)
        ssq = jnp.dot(st, e * e, preferred_element_type=jnp.float32)
        fm = 0.5 * jnp.sum(ssum * ssum - ssq, axis=0, keepdims=True)
        h = jnp.maximum(
            jnp.dot(w1_ref[...], e, preferred_element_type=jnp.float32)
            + b1_ref[...], 0.0)
        h = jnp.maximum(
            jnp.dot(w2_ref[...], h, preferred_element_type=jnp.float32)
            + b2_ref[...], 0.0)
        h = jnp.maximum(
            jnp.dot(w3_ref[...], h, preferred_element_type=jnp.float32)
            + b3_ref[...], 0.0)
        out_ref[...] = fm + h

    full = lambda shape: pl.BlockSpec(shape, lambda i: (0, 0))
    return pl.pallas_call(
        body,
        grid=(B // bs,),
        in_specs=[
            pl.BlockSpec((FD, bs), lambda i: (0, i)),
            full((128, FD)),
            full((128, 1)),
            full((16, 128)),
            full((16, 1)),
            full((2, 16)),
            full((2, 1)),
            full((D, FD)),
        ],
        out_specs=pl.BlockSpec((2, bs), lambda i: (0, i)),
        out_shape=jax.ShapeDtypeStruct((2, B), jnp.float32),
    )(embt, W1t, b1, W2t, b2, W3t, b3, St)


def kernel(x, tables, W1, b1, W2, b2, W3, b3):
    xt = x.astype(jnp.int32).T
    tables2 = tables.transpose(0, 2, 1).reshape(FD, V)
    embt = _sc_gather_t(tables2, xt)
    St = jnp.tile(jnp.eye(D, dtype=jnp.float32), (1, F))
    outt = _tc_head_t(embt, W1.T, b1.reshape(-1, 1), W2.T, b2.reshape(-1, 1),
                      W3.T, b3.reshape(-1, 1), St)
    return outt.T


# speedup vs baseline: 4.6248x; 1.4861x over previous
import functools

import jax
import jax.numpy as jnp
from jax import lax
from jax.experimental import pallas as pl
from jax.experimental.pallas import tpu as pltpu
from jax.experimental.pallas import tpu_sc as plsc

F = 26
V = 100000
D = 32
B = 16384
FD = F * D

NC = 2
NS = 16
NW = NC * NS
QUART = B // 4


def _sc_gather_t(tables2, xt):
    mesh = plsc.VectorSubcoreMesh(core_axis_name="c", subcore_axis_name="s")

    @functools.partial(
        pl.kernel,
        mesh=mesh,
        compiler_params=pltpu.CompilerParams(
            use_tc_tiling_on_sc=True, needs_layout_passes=False),
        out_type=jax.ShapeDtypeStruct((FD, B), jnp.float32),
        scratch_types=[
            pltpu.VMEM((V,), jnp.float32),
            pltpu.VMEM((QUART,), jnp.int32),
            pltpu.VMEM((QUART,), jnp.int32),
            pltpu.VMEM((QUART,), jnp.float32),
            pltpu.VMEM((QUART,), jnp.float32),
            pltpu.SemaphoreType.DMA,
            pltpu.SemaphoreType.DMA,
            pltpu.SemaphoreType.DMA,
            pltpu.SemaphoreType.DMA,
            pltpu.SemaphoreType.DMA,
        ],
    )
    def k(tab_hbm, xt_hbm, out_hbm, row_v, ib0, ib1, ob0, ob1,
          sem_row, sem_i0, sem_i1, sem_o0, sem_o1):
        wid = lax.axis_index("s") * NC + lax.axis_index("c")
        ibufs, isems = (ib0, ib1), (sem_i0, sem_i1)
        obufs, osems = (ob0, ob1), (sem_o0, sem_o1)
        pend_i = [None, None]
        pend_o = [None, None]

        U = 4
        RH = 51200
        NCHUNK = F * 4

        def idx_src(c):
            kf, q = divmod(c, 4)
            return xt_hbm.at[kf, pl.ds(q * QUART, QUART)]

        pend_i[0] = pltpu.async_copy(idx_src(0), ib0, sem_i0)

        for kf in range(F):
            r = kf * D + wid
            pltpu.async_copy(tab_hbm.at[r, :], row_v, sem_row).wait()

            for q in range(4):
                c = kf * 4 + q
                ib = ibufs[c % 2]
                pend_i[c % 2].wait()
                if c + 1 < NCHUNK:
                    nb = (c + 1) % 2
                    pend_i[nb] = pltpu.async_copy(idx_src(c + 1), ibufs[nb],
                                                  isems[nb])
                if pend_o[q % 2] is not None:
                    pend_o[q % 2].wait()
                ob = obufs[q % 2]

                def jloop(j, c3, ib=ib, ob=ob):
                    o = j * (16 * U)
                    for u in range(U):
                        oo = o + u * 16
                        vidx = ib[pl.ds(oo, 16)]
                        vals = plsc.load_gather(row_v, [vidx])
                        ob[pl.ds(oo, 16)] = vals
                    return c3

                lax.fori_loop(0, QUART // (16 * U), jloop, 0)
                pend_o[q % 2] = pltpu.async_copy(
                    ob, out_hbm.at[r, pl.ds(q * QUART, QUART)], osems[q % 2])

        pend_o[0].wait()
        pend_o[1].wait()

    return k(tables2, xt)


def _tc_head_t(embt, W1t, b1, W2t, b2, W3t, b3, St):
    bs = 2048

    def body(e_ref, w1_ref, b1_ref, w2_ref, b2_ref, w3_ref, b3_ref,
             st_ref, out_ref):
        e = e_ref[...]
        st = st_ref[...]
        ssum = jnp.dot(st, e, preferred_element_type=jnp.float32)
        ssq = jnp.dot(st, e * e, preferred_element_type=jnp.float32)
        fm = 0.5 * jnp.sum(ssum * ssum - ssq, axis=0, keepdims=True)
        h = jnp.maximum(
            jnp.dot(w1_ref[...], e, preferred_element_type=jnp.float32)
            + b1_ref[...], 0.0)
        h = jnp.maximum(
            jnp.dot(w2_ref[...], h, preferred_element_type=jnp.float32)
            + b2_ref[...], 0.0)
        h = jnp.maximum(
            jnp.dot(w3_ref[...], h, preferred_element_type=jnp.float32)
            + b3_ref[...], 0.0)
        out_ref[...] = fm + h

    full = lambda shape: pl.BlockSpec(shape, lambda i: (0, 0))
    return pl.pallas_call(
        body,
        grid=(B // bs,),
        in_specs=[
            pl.BlockSpec((FD, bs), lambda i: (0, i)),
            full((128, FD)),
            full((128, 1)),
            full((16, 128)),
            full((16, 1)),
            full((2, 16)),
            full((2, 1)),
            full((D, FD)),
        ],
        out_specs=pl.BlockSpec((2, bs), lambda i: (0, i)),
        out_shape=jax.ShapeDtypeStruct((2, B), jnp.float32),
    )(embt, W1t, b1, W2t, b2, W3t, b3, St)


def kernel(x, tables, W1, b1, W2, b2, W3, b3):
    xt = x.astype(jnp.int32).T
    tables2 = tables.transpose(0, 2, 1).reshape(FD, V)
    embt = _sc_gather_t(tables2, xt)
    St = jnp.tile(jnp.eye(D, dtype=jnp.float32), (1, F))
    outt = _tc_head_t(embt, W1.T, b1.reshape(-1, 1), W2.T, b2.reshape(-1, 1),
                      W3.T, b3.reshape(-1, 1), St)
    return outt.T


# speedup vs baseline: 5.1770x; 1.1194x over previous
import functools

import jax
import jax.numpy as jnp
from jax import lax
from jax.experimental import pallas as pl
from jax.experimental.pallas import tpu as pltpu
from jax.experimental.pallas import tpu_sc as plsc

F = 26
V = 100000
D = 32
B = 16384
FD = F * D

NC = 2
NS = 16
NW = NC * NS
QUART = B // 4


def _sc_gather_t(tables2, xt):
    mesh = plsc.VectorSubcoreMesh(core_axis_name="c", subcore_axis_name="s")

    @functools.partial(
        pl.kernel,
        mesh=mesh,
        compiler_params=pltpu.CompilerParams(
            use_tc_tiling_on_sc=True, needs_layout_passes=False),
        out_type=jax.ShapeDtypeStruct((FD, B), jnp.float32),
        scratch_types=[
            pltpu.VMEM((V,), jnp.float32),
            pltpu.VMEM((QUART,), jnp.int32),
            pltpu.VMEM((QUART,), jnp.int32),
            pltpu.VMEM((QUART,), jnp.float32),
            pltpu.VMEM((QUART,), jnp.float32),
            pltpu.SemaphoreType.DMA,
            pltpu.SemaphoreType.DMA,
            pltpu.SemaphoreType.DMA,
            pltpu.SemaphoreType.DMA,
            pltpu.SemaphoreType.DMA,
        ],
    )
    def k(tab_hbm, xt_hbm, out_hbm, row_v, ib0, ib1, ob0, ob1,
          sem_row, sem_i0, sem_i1, sem_o0, sem_o1):
        wid = lax.axis_index("s") * NC + lax.axis_index("c")
        ibufs, isems = (ib0, ib1), (sem_i0, sem_i1)
        obufs, osems = (ob0, ob1), (sem_o0, sem_o1)
        pend_i = [None, None]
        pend_o = [None, None]

        U = 4
        RH = 51200
        NCHUNK = F * 4

        def idx_src(c):
            kf, q = divmod(c, 4)
            return xt_hbm.at[kf, pl.ds(q * QUART, QUART)]

        pend_i[0] = pltpu.async_copy(idx_src(0), ib0, sem_i0)

        for kf in range(F):
            r = kf * D + wid
            pltpu.async_copy(tab_hbm.at[r, :], row_v, sem_row).wait()

            for q in range(4):
                c = kf * 4 + q
                ib = ibufs[c % 2]
                pend_i[c % 2].wait()
                if c + 1 < NCHUNK:
                    nb = (c + 1) % 2
                    pend_i[nb] = pltpu.async_copy(idx_src(c + 1), ibufs[nb],
                                                  isems[nb])
                if pend_o[q % 2] is not None:
                    pend_o[q % 2].wait()
                ob = obufs[q % 2]

                @plsc.parallel_loop(0, QUART // 16, unroll=U)
                def jloop(j, ib=ib, ob=ob):
                    oo = j * 16
                    vidx = ib[pl.ds(oo, 16)]
                    vals = plsc.load_gather(row_v, [vidx])
                    ob[pl.ds(oo, 16)] = vals
                pend_o[q % 2] = pltpu.async_copy(
                    ob, out_hbm.at[r, pl.ds(q * QUART, QUART)], osems[q % 2])

        pend_o[0].wait()
        pend_o[1].wait()

    return k(tables2, xt)


def _tc_head_t(embt, W1t, b1, W2t, b2, W3t, b3, St):
    bs = 2048

    def body(e_ref, w1_ref, b1_ref, w2_ref, b2_ref, w3_ref, b3_ref,
             st_ref, out_ref):
        e = e_ref[...]
        st = st_ref[...]
        ssum = jnp.dot(st, e, preferred_element_type=jnp.float32)
        ssq = jnp.dot(st, e * e, preferred_element_type=jnp.float32)
        fm = 0.5 * jnp.sum(ssum * ssum - ssq, axis=0, keepdims=True)
        h = jnp.maximum(
            jnp.dot(w1_ref[...], e, preferred_element_type=jnp.float32)
            + b1_ref[...], 0.0)
        h = jnp.maximum(
            jnp.dot(w2_ref[...], h, preferred_element_type=jnp.float32)
            + b2_ref[...], 0.0)
        h = jnp.maximum(
            jnp.dot(w3_ref[...], h, preferred_element_type=jnp.float32)
            + b3_ref[...], 0.0)
        out_ref[...] = fm + h

    full = lambda shape: pl.BlockSpec(shape, lambda i: (0, 0))
    return pl.pallas_call(
        body,
        grid=(B // bs,),
        in_specs=[
            pl.BlockSpec((FD, bs), lambda i: (0, i)),
            full((128, FD)),
            full((128, 1)),
            full((16, 128)),
            full((16, 1)),
            full((2, 16)),
            full((2, 1)),
            full((D, FD)),
        ],
        out_specs=pl.BlockSpec((2, bs), lambda i: (0, i)),
        out_shape=jax.ShapeDtypeStruct((2, B), jnp.float32),
    )(embt, W1t, b1, W2t, b2, W3t, b3, St)


def kernel(x, tables, W1, b1, W2, b2, W3, b3):
    xt = x.astype(jnp.int32).T
    tables2 = tables.transpose(0, 2, 1).reshape(FD, V)
    embt = _sc_gather_t(tables2, xt)
    St = jnp.tile(jnp.eye(D, dtype=jnp.float32), (1, F))
    outt = _tc_head_t(embt, W1.T, b1.reshape(-1, 1), W2.T, b2.reshape(-1, 1),
                      W3.T, b3.reshape(-1, 1), St)
    return outt.T


# speedup vs baseline: 5.1914x; 1.0028x over previous
import functools

import jax
import jax.numpy as jnp
from jax import lax
from jax.experimental import pallas as pl
from jax.experimental.pallas import tpu as pltpu
from jax.experimental.pallas import tpu_sc as plsc

F = 26
V = 100000
D = 32
B = 16384
FD = F * D

NC = 2
NS = 16
NW = NC * NS
QUART = B // 4


def _sc_gather_t(tables2, xt):
    mesh = plsc.VectorSubcoreMesh(core_axis_name="c", subcore_axis_name="s")

    @functools.partial(
        pl.kernel,
        mesh=mesh,
        compiler_params=pltpu.CompilerParams(
            use_tc_tiling_on_sc=True, needs_layout_passes=False),
        out_type=jax.ShapeDtypeStruct((FD, B), jnp.float32),
        scratch_types=[
            pltpu.VMEM((V,), jnp.float32),
            pltpu.VMEM((QUART,), jnp.int32),
            pltpu.VMEM((QUART,), jnp.int32),
            pltpu.VMEM((QUART,), jnp.float32),
            pltpu.VMEM((QUART,), jnp.float32),
            pltpu.SemaphoreType.DMA,
            pltpu.SemaphoreType.DMA,
            pltpu.SemaphoreType.DMA,
            pltpu.SemaphoreType.DMA,
            pltpu.SemaphoreType.DMA,
        ],
    )
    def k(tab_hbm, xt_hbm, out_hbm, row_v, ib0, ib1, ob0, ob1,
          sem_row, sem_i0, sem_i1, sem_o0, sem_o1):
        wid = lax.axis_index("s") * NC + lax.axis_index("c")
        ibufs, isems = (ib0, ib1), (sem_i0, sem_i1)
        obufs, osems = (ob0, ob1), (sem_o0, sem_o1)
        pend_i = [None, None]
        pend_o = [None, None]

        U = 8
        RH = 51200
        NCHUNK = F * 4

        def idx_src(c):
            kf, q = divmod(c, 4)
            return xt_hbm.at[kf, pl.ds(q * QUART, QUART)]

        pend_i[0] = pltpu.async_copy(idx_src(0), ib0, sem_i0)

        for kf in range(F):
            r = kf * D + wid
            pltpu.async_copy(tab_hbm.at[r, :], row_v, sem_row).wait()

            for q in range(4):
                c = kf * 4 + q
                ib = ibufs[c % 2]
                pend_i[c % 2].wait()
                if c + 1 < NCHUNK:
                    nb = (c + 1) % 2
                    pend_i[nb] = pltpu.async_copy(idx_src(c + 1), ibufs[nb],
                                                  isems[nb])
                if pend_o[q % 2] is not None:
                    pend_o[q % 2].wait()
                ob = obufs[q % 2]

                @plsc.parallel_loop(0, QUART // 16, unroll=U)
                def jloop(j, ib=ib, ob=ob):
                    oo = j * 16
                    vidx = ib[pl.ds(oo, 16)]
                    vals = plsc.load_gather(row_v, [vidx])
                    ob[pl.ds(oo, 16)] = vals
                pend_o[q % 2] = pltpu.async_copy(
                    ob, out_hbm.at[r, pl.ds(q * QUART, QUART)], osems[q % 2])

        pend_o[0].wait()
        pend_o[1].wait()

    return k(tables2, xt)


def _tc_head_t(embt, W1t, b1, W2t, b2, W3t, b3, St):
    bs = 2048

    def body(e_ref, w1_ref, b1_ref, w2_ref, b2_ref, w3_ref, b3_ref,
             st_ref, out_ref):
        e = e_ref[...]
        st = st_ref[...]
        ssum = jnp.dot(st, e, preferred_element_type=jnp.float32)
        ssq = jnp.dot(st, e * e, preferred_element_type=jnp.float32)
        fm = 0.5 * jnp.sum(ssum * ssum - ssq, axis=0, keepdims=True)
        h = jnp.maximum(
            jnp.dot(w1_ref[...], e, preferred_element_type=jnp.float32)
            + b1_ref[...], 0.0)
        h = jnp.maximum(
            jnp.dot(w2_ref[...], h, preferred_element_type=jnp.float32)
            + b2_ref[...], 0.0)
        h = jnp.maximum(
            jnp.dot(w3_ref[...], h, preferred_element_type=jnp.float32)
            + b3_ref[...], 0.0)
        out_ref[...] = fm + h

    full = lambda shape: pl.BlockSpec(shape, lambda i: (0, 0))
    return pl.pallas_call(
        body,
        grid=(B // bs,),
        in_specs=[
            pl.BlockSpec((FD, bs), lambda i: (0, i)),
            full((128, FD)),
            full((128, 1)),
            full((16, 128)),
            full((16, 1)),
            full((2, 16)),
            full((2, 1)),
            full((D, FD)),
        ],
        out_specs=pl.BlockSpec((2, bs), lambda i: (0, i)),
        out_shape=jax.ShapeDtypeStruct((2, B), jnp.float32),
    )(embt, W1t, b1, W2t, b2, W3t, b3, St)


def kernel(x, tables, W1, b1, W2, b2, W3, b3):
    xt = x.astype(jnp.int32).T
    tables2 = tables.transpose(0, 2, 1).reshape(FD, V)
    embt = _sc_gather_t(tables2, xt)
    St = jnp.tile(jnp.eye(D, dtype=jnp.float32), (1, F))
    outt = _tc_head_t(embt, W1.T, b1.reshape(-1, 1), W2.T, b2.reshape(-1, 1),
                      W3.T, b3.reshape(-1, 1), St)
    return outt.T
